# baseline (speedup 1.0000x reference)
import functools

import jax
import jax.numpy as jnp
import numpy as np
from jax import lax
from jax.experimental import pallas as pl
from jax.experimental.pallas import tpu as pltpu
from jax.experimental.pallas import tpu_sc as plsc

_NC = 2
_NS = 16
_NW = _NC * _NS
_L = 16

_GCHUNK = 128


@functools.lru_cache(maxsize=None)
def _build_sc_gather(vocab: int, embed: int, batch: int, tp: int):
    n_chunks = batch // tp
    bw = batch // _NW
    wpi = n_chunks // bw
    xc = bw * tp
    assert bw % _GCHUNK == 0 and embed == _L
    n_g = bw // _GCHUNK
    n_v = bw // _L

    mesh = plsc.VectorSubcoreMesh(core_axis_name="c", subcore_axis_name="s")

    @functools.partial(
        pl.kernel,
        out_type=jax.ShapeDtypeStruct((batch, embed), jnp.float32),
        mesh=mesh,
        scratch_types=[
            pltpu.VMEM((xc,), jnp.int32),
            pltpu.VMEM((n_g, _GCHUNK), jnp.int32),
            pltpu.VMEM((bw, embed), jnp.float32),
            pltpu.SemaphoreType.DMA,
        ],
        compiler_params=pltpu.CompilerParams(
            needs_layout_passes=False, use_tc_tiling_on_sc=False
        ),
    )
    def k(x_hbm, table_hbm, out_hbm, xraw_v, idx_v, rows_v, sem):
        wid = lax.axis_index("s") * _NC + lax.axis_index("c")
        i = wid // wpi
        j0 = (wid % wpi) * bw
        pltpu.sync_copy(x_hbm.at[pl.ds(j0 * tp, xc)], xraw_v)
        lanes = lax.iota(jnp.int32, _L)
        for v in range(n_v):
            offs = (lanes + v * _L) * tp + i
            vals = plsc.load_gather(xraw_v, [offs])
            idx_v[v // (_GCHUNK // _L), pl.ds((v % (_GCHUNK // _L)) * _L, _L)] = vals
        copies = [
            pltpu.async_copy(
                table_hbm.at[idx_v.at[np.int32(g)]],
                rows_v.at[pl.ds(g * _GCHUNK, _GCHUNK)],
                sem,
            )
            for g in range(n_g)
        ]
        for c in copies:
            c.wait()
        pltpu.sync_copy(rows_v, out_hbm.at[pl.ds(wid * bw, bw)])

    return k


def kernel(x, table):
    vocab, embed = table.shape
    (batch,) = x.shape
    tp = 8
    xi = x.astype(jnp.int32)
    tf = table.astype(jnp.float32)
    out = _build_sc_gather(vocab, embed, batch, tp)(xi, tf)
    return out.reshape(tp, (batch // tp) * embed)
